# baseline (speedup 1.0000x reference)
import functools

import jax
import jax.numpy as jnp
from jax import lax
from jax.experimental import pallas as pl
from jax.experimental.pallas import tpu as pltpu
from jax.experimental.pallas import tpu_sc as plsc

_SEG = 128
_K = 8
_CHUNK = _SEG * _K


def _gather_kernel(n_per_w, n_chunks, idx_hbm, table_hbm, out_hbm,
                   idx_v, rows_v, sem):
    wid = lax.axis_index("s") * 2 + lax.axis_index("c")
    base = wid * n_per_w

    def body(i, _):
        g = wid * n_chunks + i
        off = base + i * _CHUNK
        pltpu.sync_copy(idx_hbm.at[g], idx_v)
        copies = []
        for j in range(_K):
            copies.append(pltpu.async_copy(
                table_hbm.at[idx_v.at[j]],
                rows_v.at[pl.ds(j * _SEG, _SEG)],
                sem))
        for c in copies:
            c.wait()
        pltpu.sync_copy(rows_v, out_hbm.at[pl.ds(off, _CHUNK)])
        return 0

    lax.fori_loop(0, n_chunks, body, 0)


def kernel(paragraph_variable, embedding_table):
    B, H = paragraph_variable.shape
    V, D = embedding_table.shape
    N = B * H
    NW = 32
    n_per_w = N // NW
    n_chunks = n_per_w // _CHUNK

    idx = paragraph_variable.astype(jnp.int32).reshape(
        NW * n_chunks, _K, _SEG)

    mesh = plsc.VectorSubcoreMesh(core_axis_name="c", subcore_axis_name="s")
    run = pl.kernel(
        functools.partial(_gather_kernel, n_per_w, n_chunks),
        mesh=mesh,
        out_type=jax.ShapeDtypeStruct((N, D), jnp.float32),
        scratch_types=[
            pltpu.VMEM((_K, _SEG), jnp.int32),
            pltpu.VMEM((_CHUNK, D), jnp.float32),
            pltpu.SemaphoreType.DMA,
        ],
        compiler_params=pltpu.CompilerParams(use_tc_tiling_on_sc=False),
    )
    out = run(idx, embedding_table)
    return out.reshape(B, H, D)


# speedup vs baseline: 1.0161x; 1.0161x over previous
import functools

import jax
import jax.numpy as jnp
from jax import lax
from jax.experimental import pallas as pl
from jax.experimental.pallas import tpu as pltpu
from jax.experimental.pallas import tpu_sc as plsc

_SEG = 128
_K = 5
_CHUNK = _SEG * _K
_NW = 32


def _gather_kernel(n_per_w, n_groups, idx_hbm, table_hbm, out_hbm,
                   idx_v, rows0, rows1, sg0, sg1, so0, so1):
    wid = lax.axis_index("s") * 2 + lax.axis_index("c")
    base = wid * n_per_w
    pltpu.sync_copy(idx_hbm.at[wid], idx_v)

    def fire_group(g, rows, sem):
        loc = g * _CHUNK
        for j in range(_K):
            pltpu.async_copy(
                table_hbm.at[idx_v.at[pl.ds(loc + j * _SEG, _SEG)]],
                rows.at[pl.ds(j * _SEG, _SEG)],
                sem)

    def fire_out(g, rows, sem):
        pltpu.async_copy(
            rows, out_hbm.at[pl.ds(base + g * _CHUNK, _CHUNK)], sem)

    def drain_gather(rows, sem):
        pltpu.make_async_copy(
            out_hbm.at[pl.ds(base, _CHUNK)], rows, sem).wait()

    def drain_out(rows, sem):
        pltpu.make_async_copy(
            rows, out_hbm.at[pl.ds(base, _CHUNK)], sem).wait()

    fire_group(0, rows0, sg0)
    npairs = n_groups // 2

    def body(t, _):
        a = 2 * t

        @pl.when(t > 0)
        def _():
            drain_out(rows1, so1)

        fire_group(a + 1, rows1, sg1)
        drain_gather(rows0, sg0)
        fire_out(a, rows0, so0)
        drain_out(rows0, so0)

        @pl.when(t < npairs - 1)
        def _():
            fire_group(a + 2, rows0, sg0)

        drain_gather(rows1, sg1)
        fire_out(a + 1, rows1, so1)
        return 0

    lax.fori_loop(0, npairs, body, 0)
    drain_out(rows1, so1)


def kernel(paragraph_variable, embedding_table):
    B, H = paragraph_variable.shape
    V, D = embedding_table.shape
    N = B * H
    n_per_w = N // _NW
    n_groups = n_per_w // _CHUNK

    idx = paragraph_variable.astype(jnp.int32).reshape(_NW, n_per_w)

    mesh = plsc.VectorSubcoreMesh(core_axis_name="c", subcore_axis_name="s")
    run = pl.kernel(
        functools.partial(_gather_kernel, n_per_w, n_groups),
        mesh=mesh,
        out_type=jax.ShapeDtypeStruct((N, D), jnp.float32),
        scratch_types=[
            pltpu.VMEM((n_per_w,), jnp.int32),
            pltpu.VMEM((_CHUNK, D), jnp.float32),
            pltpu.VMEM((_CHUNK, D), jnp.float32),
            pltpu.SemaphoreType.DMA,
            pltpu.SemaphoreType.DMA,
            pltpu.SemaphoreType.DMA,
            pltpu.SemaphoreType.DMA,
        ],
        compiler_params=pltpu.CompilerParams(use_tc_tiling_on_sc=False),
    )
    out = run(idx, embedding_table)
    return out.reshape(B, H, D)
